# baseline (speedup 1.0000x reference)
import jax
import jax.numpy as jnp
from jax.experimental import pallas as pl
from jax.experimental.pallas import tpu as pltpu

_R = 384


def _gcn_kernel(x_ref, w0_ref, a_ref, ad_ref, b0_ref, w1_ref, b1_ref,
                w2_ref, b2_ref, out_ref, dis_s, hs_s, diag_s, *, nblk, n):
    i = pl.program_id(0)
    m = i % nblk
    base = m * _R

    @pl.when(i < nblk)
    def _phase1():
        rsum = jnp.sum(a_ref[...], axis=1, keepdims=True)
        sbase = jnp.minimum(m, nblk - 2) * _R
        db_in = a_ref[:, pl.ds(sbase, _R)]
        db_last = ad_ref[...]
        ii = jax.lax.broadcasted_iota(jnp.int32, (_R, _R), 0)
        jj = jax.lax.broadcasted_iota(jnp.int32, (_R, _R), 1)
        db = jnp.where(m == nblk - 1, db_last, db_in)
        diag = jnp.sum(jnp.where(ii == jj, db, 0.0), axis=1, keepdims=True)
        d = rsum - diag + 2.0
        dis = jnp.where(d > 0, jax.lax.rsqrt(d), 0.0)
        dis_s[pl.ds(base, _R), :] = dis
        diag_s[pl.ds(base, _R), :] = diag
        hw = jnp.dot(x_ref[...], w0_ref[...],
                     preferred_element_type=jnp.float32)
        hs_s[pl.ds(base, _R), :] = dis * hw

    @pl.when(i >= nblk)
    def _phase2():
        y = jnp.dot(a_ref[...], hs_s[:n, :],
                    preferred_element_type=jnp.float32)
        own = hs_s[pl.ds(base, _R), :]
        y = y + (2.0 - diag_s[pl.ds(base, _R), :]) * own
        h = jnp.maximum(dis_s[pl.ds(base, _R), :] * y + b0_ref[...], 0.0)
        h = jnp.maximum(
            jnp.dot(h, w1_ref[...], preferred_element_type=jnp.float32)
            + b1_ref[...], 0.0)
        o = jnp.dot(h, w2_ref[...], preferred_element_type=jnp.float32)
        out_ref[...] = jax.nn.sigmoid(o + b2_ref[...])


def kernel(x, a, W0, b0, W1, b1, W2, b2):
    import functools

    n, f = x.shape
    c = W0.shape[1]
    b0r = b0.reshape(1, c)
    b1r = b1.reshape(1, c)
    b2r = b2.reshape(1, 1)

    nblk = pl.cdiv(n, _R)
    npad = nblk * _R

    def _rowblk(i):
        m = i % nblk
        return (m, 0)

    def _diagblk(i):
        return (nblk - 1, nblk - 1)

    def _outblk(i):
        return (jnp.where(i >= nblk, i - nblk, 0), 0)

    out = pl.pallas_call(
        functools.partial(_gcn_kernel, nblk=nblk, n=n),
        grid=(2 * nblk,),
        in_specs=[
            pl.BlockSpec((_R, f), _rowblk),
            pl.BlockSpec((f, c), lambda i: (0, 0)),
            pl.BlockSpec((_R, n), _rowblk),
            pl.BlockSpec((_R, _R), _diagblk),
            pl.BlockSpec((1, c), lambda i: (0, 0)),
            pl.BlockSpec((c, c), lambda i: (0, 0)),
            pl.BlockSpec((1, c), lambda i: (0, 0)),
            pl.BlockSpec((c, 1), lambda i: (0, 0)),
            pl.BlockSpec((1, 1), lambda i: (0, 0)),
        ],
        out_specs=pl.BlockSpec((_R, 1), _outblk),
        out_shape=jax.ShapeDtypeStruct((n, 1), jnp.float32),
        scratch_shapes=[
            pltpu.VMEM((npad, 1), jnp.float32),
            pltpu.VMEM((npad, c), jnp.float32),
            pltpu.VMEM((npad, 1), jnp.float32),
        ],
    )(x, W0, a, a, b0r, W1, b1r, W2, b2r)
    return out


# speedup vs baseline: 1.0176x; 1.0176x over previous
import jax
import jax.numpy as jnp
from jax.experimental import pallas as pl
from jax.experimental.pallas import tpu as pltpu

_R = 512


def _gcn_kernel(x_ref, w0_ref, a_ref, ad_ref, b0_ref, w1_ref, b1_ref,
                w2_ref, b2_ref, out_ref, dis_s, hs_s, diag_s, *, nblk, n):
    i = pl.program_id(0)
    m = i % nblk
    base = m * _R

    @pl.when(i < nblk)
    def _phase1():
        rsum = jnp.sum(a_ref[...], axis=1, keepdims=True)
        sbase = jnp.minimum(m, nblk - 2) * _R
        db_in = a_ref[:, pl.ds(sbase, _R)]
        db_last = ad_ref[...]
        ii = jax.lax.broadcasted_iota(jnp.int32, (_R, _R), 0)
        jj = jax.lax.broadcasted_iota(jnp.int32, (_R, _R), 1)
        db = jnp.where(m == nblk - 1, db_last, db_in)
        diag = jnp.sum(jnp.where(ii == jj, db, 0.0), axis=1, keepdims=True)
        d = rsum - diag + 2.0
        dis = jnp.where(d > 0, jax.lax.rsqrt(d), 0.0)
        dis_s[pl.ds(base, _R), :] = dis
        diag_s[pl.ds(base, _R), :] = diag
        hw = jnp.dot(x_ref[...], w0_ref[...],
                     preferred_element_type=jnp.float32)
        hs_s[pl.ds(base, _R), :] = dis * hw

    @pl.when(i >= nblk)
    def _phase2():
        y = jnp.dot(a_ref[...], hs_s[:n, :],
                    preferred_element_type=jnp.float32)
        own = hs_s[pl.ds(base, _R), :]
        y = y + (2.0 - diag_s[pl.ds(base, _R), :]) * own
        h = jnp.maximum(dis_s[pl.ds(base, _R), :] * y + b0_ref[...], 0.0)
        h = jnp.maximum(
            jnp.dot(h, w1_ref[...], preferred_element_type=jnp.float32)
            + b1_ref[...], 0.0)
        o = jnp.dot(h, w2_ref[...], preferred_element_type=jnp.float32)
        out_ref[...] = jax.nn.sigmoid(o + b2_ref[...])


def kernel(x, a, W0, b0, W1, b1, W2, b2):
    import functools

    n, f = x.shape
    c = W0.shape[1]
    b0r = b0.reshape(1, c)
    b1r = b1.reshape(1, c)
    b2r = b2.reshape(1, 1)

    nblk = pl.cdiv(n, _R)
    npad = nblk * _R

    def _rowblk(i):
        m = i % nblk
        return (m, 0)

    def _xblk(i):
        return (jnp.minimum(i, nblk - 1), 0)

    def _diagblk(i):
        return (nblk - 1, nblk - 1)

    def _outblk(i):
        return (jnp.where(i >= nblk, i - nblk, 0), 0)

    out = pl.pallas_call(
        functools.partial(_gcn_kernel, nblk=nblk, n=n),
        grid=(2 * nblk,),
        in_specs=[
            pl.BlockSpec((_R, f), _xblk),
            pl.BlockSpec((f, c), lambda i: (0, 0)),
            pl.BlockSpec((_R, n), _rowblk),
            pl.BlockSpec((_R, _R), _diagblk),
            pl.BlockSpec((1, c), lambda i: (0, 0)),
            pl.BlockSpec((c, c), lambda i: (0, 0)),
            pl.BlockSpec((1, c), lambda i: (0, 0)),
            pl.BlockSpec((c, 1), lambda i: (0, 0)),
            pl.BlockSpec((1, 1), lambda i: (0, 0)),
        ],
        out_specs=pl.BlockSpec((_R, 1), _outblk),
        out_shape=jax.ShapeDtypeStruct((n, 1), jnp.float32),
        scratch_shapes=[
            pltpu.VMEM((npad, 1), jnp.float32),
            pltpu.VMEM((npad, c), jnp.float32),
            pltpu.VMEM((npad, 1), jnp.float32),
        ],
    )(x, W0, a, a, b0r, W1, b1r, W2, b2r)
    return out
